# baseline (speedup 1.0000x reference)
import functools

import jax
import jax.numpy as jnp
from jax import lax
from jax.experimental import pallas as pl
from jax.experimental.pallas import tpu as pltpu
from jax.experimental.pallas import tpu_sc as plsc

_VOCAB = 1000000
_DIM = 32
_B = 4096
_C = 4096


@functools.lru_cache(maxsize=None)
def _make_sc_gather(V, D, B, C):
    NC, NS = 2, 16
    NW = NC * NS
    b_per_w = B // NW
    c_per_w = C // NW
    DH = D // 8
    NB = 2
    NBUF = 8
    mesh = plsc.VectorSubcoreMesh(core_axis_name="c", subcore_axis_name="s")

    @functools.partial(
        pl.kernel,
        mesh=mesh,
        out_type=[
            jax.ShapeDtypeStruct((DH, 8, B), jnp.float32),
            jax.ShapeDtypeStruct((DH, 8, C), jnp.float32),
        ],
        scratch_types=[
            pltpu.VMEM((b_per_w + 2 * NB,), jnp.int32),
            pltpu.VMEM((b_per_w + 2 * NB,), jnp.int32),
            pltpu.VMEM((c_per_w + 2 * NB,), jnp.int32),
            pltpu.VMEM((c_per_w + 2 * NB,), jnp.int32),
            pltpu.VMEM((NBUF, NB, DH, 8, 128), jnp.float32),
            pltpu.VMEM((DH, 8, b_per_w), jnp.float32),
            pltpu.VMEM((DH, 8, c_per_w), jnp.float32),
            pltpu.SemaphoreType.DMA,
            pltpu.SemaphoreType.DMA,
            pltpu.SemaphoreType.DMA,
            pltpu.SemaphoreType.DMA,
            pltpu.SemaphoreType.DMA,
            pltpu.SemaphoreType.DMA,
            pltpu.SemaphoreType.DMA,
            pltpu.SemaphoreType.DMA,
        ],
    )
    def gather_k(winT_hbm, ctcol_hbm, clane_hbm, woutT_hbm, xtcol_hbm,
                 xlane_hbm, outcT_hbm, outxT_hbm,
                 ctcol_v, clane_v, xtcol_v, xlane_v,
                 slots_v, cacc_v, xacc_v, sem0, sem1, sem2, sem3, sem4, sem5, sem6, sem7):
        wid = lax.axis_index("s") * NC + lax.axis_index("c")
        cbase = wid * b_per_w
        xbase = wid * c_per_w
        pltpu.sync_copy(ctcol_hbm.at[pl.ds(cbase, b_per_w)],
                        ctcol_v.at[pl.ds(0, b_per_w)])
        pltpu.sync_copy(clane_hbm.at[pl.ds(cbase, b_per_w)],
                        clane_v.at[pl.ds(0, b_per_w)])
        pltpu.sync_copy(xtcol_hbm.at[pl.ds(xbase, c_per_w)],
                        xtcol_v.at[pl.ds(0, c_per_w)])
        pltpu.sync_copy(xlane_hbm.at[pl.ds(xbase, c_per_w)],
                        xlane_v.at[pl.ds(0, c_per_w)])
        d16 = lax.iota(jnp.int32, 16)
        sems = (sem0, sem1, sem2, sem3, sem4, sem5, sem6, sem7)

        def gather_table(tab_hbm, tcol_v, lane_v, acc_v, n_words):
            nbatch = n_words // NB

            def fire(slot, sem, off16):
                tcol = tcol_v[pl.ds(off16, 16)]
                for b in range(NB):
                    pltpu.async_copy(
                        tab_hbm.at[:, :, pl.ds(pl.multiple_of(tcol[b], 128), 128)],
                        slots_v.at[slot, b], sem,
                    )

            def drain(slot, sem):
                for b in range(NB):
                    pltpu.make_async_copy(
                        tab_hbm.at[:, :, pl.ds(0, 128)],
                        slots_v.at[slot, b], sem,
                    ).wait()

            def extract(slot, p, off16, colg16):
                lane = lane_v[pl.ds(off16, 16)]
                for h in range(DH):
                    for s in range(8):
                        cur = acc_v[h, s, pl.ds(colg16, 16)]
                        for b in range(NB):
                            lb = lane[b]
                            lc16 = pl.multiple_of((lb // 16) * 16, 16)
                            li = jnp.broadcast_to(lb - lc16, (16,))
                            v = slots_v[slot, b, h, s, pl.ds(lc16, 16)]
                            gv = lax.gather(
                                v, li[:, None],
                                lax.GatherDimensionNumbers(
                                    offset_dims=(), collapsed_slice_dims=(0,),
                                    start_index_map=(0,)),
                                (1,),
                                mode=lax.GatherScatterMode.PROMISE_IN_BOUNDS)
                            cur = jnp.where(d16 == p * NB + b, gv, cur)
                        acc_v[h, s, pl.ds(colg16, 16)] = cur

            for q in range(NBUF):
                fire(q, sems[q], q * NB)

            def body(u, _):
                colg16 = pl.multiple_of(u * 16, 16)
                for p in range(NBUF):
                    t = NBUF * u + p
                    drain(p, sems[p])
                    extract(p, p, t * NB, colg16)

                    @pl.when(t + NBUF < nbatch)
                    def _():
                        fire(p, sems[p], (t + NBUF) * NB)
                return 0

            lax.fori_loop(0, nbatch // NBUF, body, 0)

        gather_table(winT_hbm, ctcol_v, clane_v, cacc_v, b_per_w)
        gather_table(woutT_hbm, xtcol_v, xlane_v, xacc_v, c_per_w)
        pltpu.sync_copy(cacc_v, outcT_hbm.at[:, :, pl.ds(cbase, b_per_w)])
        pltpu.sync_copy(xacc_v, outxT_hbm.at[:, :, pl.ds(xbase, c_per_w)])

    return gather_k


def _mm_body(cvT_ref, xvT_ref, out_ref):
    out_ref[...] = lax.dot_general(
        cvT_ref[...], xvT_ref[...],
        (((0,), (0,)), ((), ())),
        preferred_element_type=jnp.float32,
    )


def _matmul(cvT, xvT):
    BM = 512
    BN = 4096
    grid = (_B // BM, _C // BN)
    return pl.pallas_call(
        _mm_body,
        grid=grid,
        in_specs=[
            pl.BlockSpec((_DIM, BM), lambda i, j: (0, i)),
            pl.BlockSpec((_DIM, BN), lambda i, j: (0, j)),
        ],
        out_specs=pl.BlockSpec((BM, BN), lambda i, j: (i, j)),
        out_shape=jax.ShapeDtypeStruct((_B, _C), jnp.float32),
    )(cvT, xvT)


def kernel(center_words, all_context_words, W_in, W_out):
    cidx = center_words.astype(jnp.int32)
    xidx = all_context_words.astype(jnp.int32)
    ctcol = (cidx // 128) * 128
    clane = cidx % 128
    xtcol = (xidx // 128) * 128
    xlane = xidx % 128
    cvT3, xvT3 = _make_sc_gather(_VOCAB, _DIM, _B, _C)(
        W_in.T.reshape(_DIM // 8, 8, _VOCAB), ctcol, clane,
        W_out.T.reshape(_DIM // 8, 8, _VOCAB), xtcol, xlane)
    cvT = cvT3.reshape(_DIM, _B)
    xvT = xvT3.reshape(_DIM, _C)
    return _matmul(cvT, xvT)


# speedup vs baseline: 1.1617x; 1.1617x over previous
import functools

import jax
import jax.numpy as jnp
from jax import lax
from jax.experimental import pallas as pl
from jax.experimental.pallas import tpu as pltpu
from jax.experimental.pallas import tpu_sc as plsc

_VOCAB = 1000000
_DIM = 32
_B = 4096
_C = 4096


@functools.lru_cache(maxsize=None)
def _make_sc_gather(V, D, B, C):
    NC, NS = 2, 16
    NW = NC * NS
    b_per_w = B // NW
    c_per_w = C // NW
    DH = D // 8
    NB = 4
    NBUF = 4
    mesh = plsc.VectorSubcoreMesh(core_axis_name="c", subcore_axis_name="s")

    @functools.partial(
        pl.kernel,
        mesh=mesh,
        out_type=[
            jax.ShapeDtypeStruct((DH, 8, B), jnp.float32),
            jax.ShapeDtypeStruct((DH, 8, C), jnp.float32),
        ],
        scratch_types=[
            pltpu.VMEM((b_per_w + 2 * NB,), jnp.int32),
            pltpu.VMEM((b_per_w + 2 * NB,), jnp.int32),
            pltpu.VMEM((c_per_w + 2 * NB,), jnp.int32),
            pltpu.VMEM((c_per_w + 2 * NB,), jnp.int32),
            pltpu.VMEM((NBUF, NB, DH, 8, 128), jnp.float32),
            pltpu.VMEM((DH, 8, b_per_w), jnp.float32),
            pltpu.VMEM((DH, 8, c_per_w), jnp.float32),
            pltpu.SemaphoreType.DMA,
            pltpu.SemaphoreType.DMA,
            pltpu.SemaphoreType.DMA,
            pltpu.SemaphoreType.DMA,
        ],
    )
    def gather_k(winT_hbm, ctcol_hbm, clane_hbm, woutT_hbm, xtcol_hbm,
                 xlane_hbm, outcT_hbm, outxT_hbm,
                 ctcol_v, clane_v, xtcol_v, xlane_v,
                 slots_v, cacc_v, xacc_v, sem0, sem1, sem2, sem3):
        wid = lax.axis_index("s") * NC + lax.axis_index("c")
        cbase = wid * b_per_w
        xbase = wid * c_per_w
        pltpu.sync_copy(ctcol_hbm.at[pl.ds(cbase, b_per_w)],
                        ctcol_v.at[pl.ds(0, b_per_w)])
        pltpu.sync_copy(clane_hbm.at[pl.ds(cbase, b_per_w)],
                        clane_v.at[pl.ds(0, b_per_w)])
        pltpu.sync_copy(xtcol_hbm.at[pl.ds(xbase, c_per_w)],
                        xtcol_v.at[pl.ds(0, c_per_w)])
        pltpu.sync_copy(xlane_hbm.at[pl.ds(xbase, c_per_w)],
                        xlane_v.at[pl.ds(0, c_per_w)])
        d16 = lax.iota(jnp.int32, 16)
        sems = (sem0, sem1, sem2, sem3)

        def gather_table(tab_hbm, tcol_v, lane_v, acc_v, n_words):
            nbatch = n_words // NB

            def fire(slot, sem, off16):
                tcol = tcol_v[pl.ds(off16, 16)]
                for b in range(NB):
                    pltpu.async_copy(
                        tab_hbm.at[:, :, pl.ds(pl.multiple_of(tcol[b], 128), 128)],
                        slots_v.at[slot, b], sem,
                    )

            def drain(slot, sem):
                for b in range(NB):
                    pltpu.make_async_copy(
                        tab_hbm.at[:, :, pl.ds(0, 128)],
                        slots_v.at[slot, b], sem,
                    ).wait()

            def extract(slot, p, off16, colg16):
                lane = lane_v[pl.ds(off16, 16)]
                for h in range(DH):
                    for s in range(8):
                        cur = acc_v[h, s, pl.ds(colg16, 16)]
                        for b in range(NB):
                            lb = lane[b]
                            lc16 = pl.multiple_of((lb // 16) * 16, 16)
                            li = jnp.broadcast_to(lb - lc16, (16,))
                            v = slots_v[slot, b, h, s, pl.ds(lc16, 16)]
                            gv = lax.gather(
                                v, li[:, None],
                                lax.GatherDimensionNumbers(
                                    offset_dims=(), collapsed_slice_dims=(0,),
                                    start_index_map=(0,)),
                                (1,),
                                mode=lax.GatherScatterMode.PROMISE_IN_BOUNDS)
                            cur = jnp.where(d16 == p * NB + b, gv, cur)
                        acc_v[h, s, pl.ds(colg16, 16)] = cur

            for q in range(NBUF):
                fire(q, sems[q], q * NB)

            def body(u, _):
                colg16 = pl.multiple_of(u * 16, 16)
                for p in range(NBUF):
                    t = NBUF * u + p
                    drain(p, sems[p])
                    extract(p, p, t * NB, colg16)

                    @pl.when(t + NBUF < nbatch)
                    def _():
                        fire(p, sems[p], (t + NBUF) * NB)
                return 0

            lax.fori_loop(0, nbatch // NBUF, body, 0)

        gather_table(winT_hbm, ctcol_v, clane_v, cacc_v, b_per_w)
        gather_table(woutT_hbm, xtcol_v, xlane_v, xacc_v, c_per_w)
        pltpu.sync_copy(cacc_v, outcT_hbm.at[:, :, pl.ds(cbase, b_per_w)])
        pltpu.sync_copy(xacc_v, outxT_hbm.at[:, :, pl.ds(xbase, c_per_w)])

    return gather_k


def _mm_body(cvT_ref, xvT_ref, out_ref):
    out_ref[...] = lax.dot_general(
        cvT_ref[...], xvT_ref[...],
        (((0,), (0,)), ((), ())),
        preferred_element_type=jnp.float32,
    )


def _matmul(cvT, xvT):
    BM = 512
    BN = 4096
    grid = (_B // BM, _C // BN)
    return pl.pallas_call(
        _mm_body,
        grid=grid,
        in_specs=[
            pl.BlockSpec((_DIM, BM), lambda i, j: (0, i)),
            pl.BlockSpec((_DIM, BN), lambda i, j: (0, j)),
        ],
        out_specs=pl.BlockSpec((BM, BN), lambda i, j: (i, j)),
        out_shape=jax.ShapeDtypeStruct((_B, _C), jnp.float32),
    )(cvT, xvT)


def kernel(center_words, all_context_words, W_in, W_out):
    cidx = center_words.astype(jnp.int32)
    xidx = all_context_words.astype(jnp.int32)
    ctcol = (cidx // 128) * 128
    clane = cidx % 128
    xtcol = (xidx // 128) * 128
    xlane = xidx % 128
    cvT3, xvT3 = _make_sc_gather(_VOCAB, _DIM, _B, _C)(
        W_in.T.reshape(_DIM // 8, 8, _VOCAB), ctcol, clane,
        W_out.T.reshape(_DIM // 8, 8, _VOCAB), xtcol, xlane)
    cvT = cvT3.reshape(_DIM, _B)
    xvT = xvT3.reshape(_DIM, _C)
    return _matmul(cvT, xvT)
